# baseline (speedup 1.0000x reference)
import functools

import jax
import jax.numpy as jnp
from jax import lax
from jax.experimental import pallas as pl
from jax.experimental.pallas import tpu as pltpu
from jax.experimental.pallas import tpu_sc as plsc

N_NODES = 10000
D_FEAT = 128
_NC = 2
_NS = 16
_NW = _NC * _NS
_ACTIVE = 25
_ROWS_PER_W = 400

_mesh = plsc.VectorSubcoreMesh(core_axis_name="c", subcore_axis_name="s")


@functools.partial(
    pl.kernel,
    mesh=_mesh,
    out_type=jax.ShapeDtypeStruct((_NW, 16), jnp.float32),
    scratch_types=[
        pltpu.VMEM((_ROWS_PER_W, D_FEAT), jnp.float32),
        pltpu.VMEM((16,), jnp.float32),
    ],
)
def _reduce_kernel(x_hbm, out_hbm, buf, vec):
    cid = lax.axis_index("c")
    sid = lax.axis_index("s")
    wid = sid * _NC + cid

    vec[...] = jnp.zeros((16,), jnp.float32)

    @pl.when(wid < _ACTIVE)
    def _compute():
        base = wid * _ROWS_PER_W
        pltpu.sync_copy(x_hbm.at[pl.ds(base, _ROWS_PER_W)], buf)

        def body(i, acc):
            return acc + buf[i, 0:16]

        acc = lax.fori_loop(0, _ROWS_PER_W, body, jnp.zeros((16,), jnp.float32))
        vec[...] = acc * jnp.float32(1.0 / (4 * N_NODES))

    pltpu.sync_copy(vec, out_hbm.at[wid])


def kernel(x, edge_index, edge_attr):
    partials = _reduce_kernel(x)
    col = jnp.sum(partials[:, :8], axis=0)
    v_x = col[0] + col[2] + col[4] + col[6]
    v_y = col[1] + col[3] + col[5] + col[7]
    return jnp.stack([v_x, v_y])


# speedup vs baseline: 8.2446x; 8.2446x over previous
import jax
import jax.numpy as jnp
from jax import lax
from jax.experimental import pallas as pl
from jax.experimental.pallas import tpu as pltpu

N_NODES = 10000
D_FEAT = 128


def _reduce_body(x_ref, o_ref):
    ones = jnp.ones((1, N_NODES), jnp.float32)
    colsum = lax.dot_general(
        ones, x_ref[...], (((1,), (0,)), ((), ())),
        preferred_element_type=jnp.float32)
    lane = lax.broadcasted_iota(jnp.int32, (1, D_FEAT), 1)
    even = (lane < 8) & (lane % 2 == 0)
    odd = (lane < 8) & (lane % 2 == 1)
    scale = jnp.float32(1.0 / (4 * N_NODES))
    o_ref[0] = jnp.sum(jnp.where(even, colsum, 0.0)) * scale
    o_ref[1] = jnp.sum(jnp.where(odd, colsum, 0.0)) * scale


_reduce = pl.pallas_call(
    _reduce_body,
    out_shape=jax.ShapeDtypeStruct((2,), jnp.float32),
    out_specs=pl.BlockSpec(memory_space=pltpu.SMEM),
)


def kernel(x, edge_index, edge_attr):
    return _reduce(x)


# speedup vs baseline: 8.7901x; 1.0662x over previous
import jax
import jax.numpy as jnp
from jax import lax
from jax.experimental import pallas as pl
from jax.experimental.pallas import tpu as pltpu

N_NODES = 10000
D_FEAT = 128
_N_CHAINS = 10
_H = N_NODES // _N_CHAINS


def _reduce_body(x_ref, o_ref):
    acc = None
    for k in range(_N_CHAINS):
        blk = x_ref[k * _H:(k + 1) * _H, :].reshape(_H // 8, 8, D_FEAT)
        s = jnp.sum(blk, axis=0)
        acc = s if acc is None else acc + s
    colsum = jnp.sum(acc, axis=0, keepdims=True)
    lane = lax.broadcasted_iota(jnp.int32, (1, D_FEAT), 1)
    even = (lane < 8) & (lane % 2 == 0)
    odd = (lane < 8) & (lane % 2 == 1)
    scale = jnp.float32(1.0 / (4 * N_NODES))
    o_ref[0] = jnp.sum(jnp.where(even, colsum, 0.0)) * scale
    o_ref[1] = jnp.sum(jnp.where(odd, colsum, 0.0)) * scale


_reduce = pl.pallas_call(
    _reduce_body,
    out_shape=jax.ShapeDtypeStruct((2,), jnp.float32),
    out_specs=pl.BlockSpec(memory_space=pltpu.SMEM),
)


def kernel(x, edge_index, edge_attr):
    return _reduce(x)
